# baseline (speedup 1.0000x reference)
import jax
import jax.numpy as jnp
from jax import lax
from jax.experimental import pallas as pl
from jax.experimental.pallas import tpu as pltpu
from jax.experimental.pallas import tpu_sc as plsc

_N = 10000
_E = 320000
_D = 128
_K = 10
_ALPHA = 0.1

_NC = 2
_NS = 16
_DH = _D // _NC
_NP = 10240
_RPT = _NP // _NS
_RC = 64
_NRC = _RPT // _RC
_C = 128
_NCHUNK = 160
_EPAD = _NS * _NCHUNK * _C
_PAD_IDX = _N
_L = 16


def _nr_rsqrt(x):
    i = lax.bitcast_convert_type(x, jnp.int32)
    i = jnp.int32(0x5F3759DF) - (i >> 1)
    y = lax.bitcast_convert_type(i, jnp.float32)
    for _ in range(3):
        y = y * (1.5 - 0.5 * x * y * y)
    return y


def _appnp_body(feat, srci, dsti, out, u_hbm, acc_sp, src_v, dst_v,
                y_v, buf0, buf1, buf2, buf3, zbuf,
                gs0, gs1, gs2, gs3, ss0, ss1, ss2, ss3):
    c = lax.axis_index("c")
    s = lax.axis_index("s")
    row0 = s * _RPT

    pltpu.sync_copy(srci.at[s], src_v)
    pltpu.sync_copy(dsti.at[s], dst_v)
    ubase = c * _NP

    def _offs(j, _):
        for k in range(_C // _L):
            src_v[j, pl.ds(_L * k, _L)] = src_v[j, pl.ds(_L * k, _L)] + ubase
        return 0
    lax.fori_loop(0, _NCHUNK, _offs, 0)

    def _zrow(r, _):
        z = jnp.zeros((_L,), jnp.float32)
        for k in range(_DH // _L):
            zbuf[r, pl.ds(_L * k, _L)] = z
        return 0
    lax.fori_loop(0, _RC, _zrow, 0)

    def _zacc(i, _):
        pltpu.sync_copy(zbuf, acc_sp.at[pl.ds(row0 + i * _RC, _RC)])
        return 0
    lax.fori_loop(0, _NRC, _zacc, 0)

    def _ones(r, _):
        o = jnp.full((_L,), 1.0, jnp.float32)
        for k in range(_DH // _L):
            buf0[r, pl.ds(_L * k, _L)] = o
        return 0
    lax.fori_loop(0, _C, _ones, 0)
    plsc.subcore_barrier()

    def _deg(t, _):
        for p in range(4):
            j = 4 * t + p
            @pl.when(t > 0)
            def _w():
                pltpu.make_async_copy(buf0, acc_sp.at[dst_v.at[j - 4]],
                                      (ss0, ss1, ss2, ss3)[p]).wait()
            pltpu.async_copy(buf0, acc_sp.at[dst_v.at[j]],
                             (ss0, ss1, ss2, ss3)[p], add=True)
        return 0
    lax.fori_loop(0, _NCHUNK // 4, _deg, 0)
    for p in range(4):
        pltpu.make_async_copy(buf0, acc_sp.at[dst_v.at[_NCHUNK - 4 + p]],
                              (ss0, ss1, ss2, ss3)[p]).wait()
    plsc.subcore_barrier()

    def _ab_chunk(ci, _):
        r0 = row0 + ci * _RC
        pltpu.sync_copy(acc_sp.at[pl.ds(r0, _RC)], buf0.at[pl.ds(0, _RC)])
        pltpu.sync_copy(feat.at[c].at[pl.ds(r0, _RC)], buf1.at[pl.ds(0, _RC)])

        def _row(r, _2):
            lr = ci * _RC + r
            d = buf0[r, pl.ds(0, _L)]
            y = _nr_rsqrt(jnp.maximum(d, 1.0))
            y_v[lr, pl.ds(0, _L)] = y
            for k in range(_DH // _L):
                h0 = buf1[r, pl.ds(_L * k, _L)]
                buf0[r, pl.ds(_L * k, _L)] = y * h0
            return 0
        lax.fori_loop(0, _RC, _row, 0)
        pltpu.sync_copy(buf0.at[pl.ds(0, _RC)], u_hbm.at[pl.ds(ubase + r0, _RC)])
        pltpu.sync_copy(zbuf, acc_sp.at[pl.ds(r0, _RC)])
        return 0
    lax.fori_loop(0, _NRC, _ab_chunk, 0)
    plsc.subcore_barrier()

    bufs = (buf0, buf1, buf2, buf3)
    gss = (gs0, gs1, gs2, gs3)
    sss = (ss0, ss1, ss2, ss3)

    def _propagate():
        pltpu.async_copy(u_hbm.at[src_v.at[0]], buf0, gs0)

        def _pair(t, _):
            j0 = 2 * t
            pltpu.make_async_copy(u_hbm.at[src_v.at[j0]], buf0, gs0).wait()

            @pl.when(j0 > 0)
            def _w():
                pltpu.make_async_copy(buf1, acc_sp.at[dst_v.at[j0 - 1]],
                                      ss1).wait()
            pltpu.async_copy(u_hbm.at[src_v.at[j0 + 1]], buf1, gs1)
            pltpu.async_copy(buf0, acc_sp.at[dst_v.at[j0]], ss0, add=True)
            pltpu.make_async_copy(u_hbm.at[src_v.at[j0 + 1]], buf1, gs1).wait()
            pltpu.make_async_copy(buf0, acc_sp.at[dst_v.at[j0]], ss0).wait()

            @pl.when(j0 + 2 < _NCHUNK)
            def _g():
                pltpu.async_copy(u_hbm.at[src_v.at[j0 + 2]], buf0, gs0)
            pltpu.async_copy(buf1, acc_sp.at[dst_v.at[j0 + 1]], ss1, add=True)
            return 0
        lax.fori_loop(0, _NCHUNK // 2, _pair, 0)
        pltpu.make_async_copy(buf1, acc_sp.at[dst_v.at[_NCHUNK - 1]],
                              ss1).wait()

    def _dense():
        def _start_loads(i, rbuf, gsa, gsf):
            r0 = row0 + i * _RC
            pltpu.async_copy(acc_sp.at[pl.ds(r0, _RC)],
                             rbuf.at[pl.ds(0, _RC)], gsa)
            pltpu.async_copy(feat.at[c].at[pl.ds(r0, _RC)],
                             rbuf.at[pl.ds(_RC, _RC)], gsf)

        def _wait_loads(i, rbuf, gsa, gsf):
            r0 = row0 + i * _RC
            pltpu.make_async_copy(acc_sp.at[pl.ds(r0, _RC)],
                                  rbuf.at[pl.ds(0, _RC)], gsa).wait()
            pltpu.make_async_copy(feat.at[c].at[pl.ds(r0, _RC)],
                                  rbuf.at[pl.ds(_RC, _RC)], gsf).wait()

        def _start_writes(i, rbuf, ssu, ssz):
            r0 = row0 + i * _RC
            pltpu.async_copy(rbuf.at[pl.ds(0, _RC)],
                             u_hbm.at[pl.ds(ubase + r0, _RC)], ssu)
            pltpu.async_copy(zbuf, acc_sp.at[pl.ds(r0, _RC)], ssz)

        def _wait_writes(i, rbuf, ssu, ssz):
            r0 = row0 + i * _RC
            pltpu.make_async_copy(rbuf.at[pl.ds(0, _RC)],
                                  u_hbm.at[pl.ds(ubase + r0, _RC)], ssu).wait()
            pltpu.make_async_copy(zbuf, acc_sp.at[pl.ds(r0, _RC)], ssz).wait()

        def _compute(i, rbuf):
            def _row(r, _2):
                lr = i * _RC + r
                y = y_v[lr, pl.ds(0, _L)]
                a = (1.0 - _ALPHA) * (y * y)
                for k in range(_DH // _L):
                    acc16 = rbuf[r, pl.ds(_L * k, _L)]
                    b16 = _ALPHA * (y * rbuf[_RC + r, pl.ds(_L * k, _L)])
                    rbuf[r, pl.ds(_L * k, _L)] = a * acc16 + b16
                return 0
            lax.fori_loop(0, _RC, _row, 0)

        _start_loads(0, buf0, gs0, gs2)

        def _pair2(t, _):
            for p in range(2):
                i = 2 * t + p
                rbuf = bufs[p]
                gsa, gsf = (gs0, gs2) if p == 0 else (gs1, gs3)
                ssu, ssz = (ss0, ss2) if p == 0 else (ss1, ss3)
                orb = bufs[1 - p]
                ogsa, ogsf = (gs1, gs3) if p == 0 else (gs0, gs2)
                ossu, ossz = (ss1, ss3) if p == 0 else (ss0, ss2)
                @pl.when(i >= 1)
                def _w():
                    _wait_writes(i - 1, orb, ossu, ossz)

                @pl.when(i + 1 < _NRC)
                def _l():
                    _start_loads(i + 1, orb, ogsa, ogsf)
                _wait_loads(i, rbuf, gsa, gsf)
                _compute(i, rbuf)
                _start_writes(i, rbuf, ssu, ssz)
            return 0
        lax.fori_loop(0, _NRC // 2, _pair2, 0)
        _wait_writes(_NRC - 1, buf1, ss1, ss3)

    def _layer(l, _):
        _propagate()
        plsc.subcore_barrier()
        _dense()
        plsc.subcore_barrier()
        return 0
    lax.fori_loop(0, _K - 1, _layer, 0)

    _propagate()
    plsc.subcore_barrier()

    def _out_chunk(ci, _):
        r0 = row0 + ci * _RC
        pltpu.sync_copy(acc_sp.at[pl.ds(r0, _RC)], buf0.at[pl.ds(0, _RC)])
        pltpu.sync_copy(feat.at[c].at[pl.ds(r0, _RC)], buf1.at[pl.ds(0, _RC)])

        def _row(r, _2):
            lr = ci * _RC + r
            y = y_v[lr, pl.ds(0, _L)]
            for k in range(_DH // _L):
                acc16 = buf0[r, pl.ds(_L * k, _L)]
                h016 = buf1[r, pl.ds(_L * k, _L)]
                buf0[r, pl.ds(_L * k, _L)] = ((1.0 - _ALPHA) * (y * acc16)
                                              + _ALPHA * h016)
            return 0
        lax.fori_loop(0, _RC, _row, 0)
        pltpu.sync_copy(buf0.at[pl.ds(0, _RC)], out.at[c].at[pl.ds(r0, _RC)])
        return 0
    lax.fori_loop(0, _NRC, _out_chunk, 0)


_launch = pl.kernel(
    _appnp_body,
    out_type=jax.ShapeDtypeStruct((_NC, _NP, _DH), jnp.float32),
    mesh=plsc.VectorSubcoreMesh(core_axis_name="c", subcore_axis_name="s"),
    compiler_params=pltpu.CompilerParams(use_tc_tiling_on_sc=False),
    scratch_types=[
        pltpu.HBM((_NC * _NP, _DH), jnp.float32),
        pltpu.VMEM_SHARED((_NP, _DH), jnp.float32),
        pltpu.VMEM((_NCHUNK, _C), jnp.int32),
        pltpu.VMEM((_NCHUNK, _C), jnp.int32),
        pltpu.VMEM((_RPT, _L), jnp.float32),
        pltpu.VMEM((_C, _DH), jnp.float32),
        pltpu.VMEM((_C, _DH), jnp.float32),
        pltpu.VMEM((_C, _DH), jnp.float32),
        pltpu.VMEM((_C, _DH), jnp.float32),
        pltpu.VMEM((_RC, _DH), jnp.float32),
        pltpu.SemaphoreType.DMA,
        pltpu.SemaphoreType.DMA,
        pltpu.SemaphoreType.DMA,
        pltpu.SemaphoreType.DMA,
        pltpu.SemaphoreType.DMA,
        pltpu.SemaphoreType.DMA,
        pltpu.SemaphoreType.DMA,
        pltpu.SemaphoreType.DMA,
    ],
)


def kernel(features, edge_index):
    src = edge_index[0]
    dst = edge_index[1]
    srcp = jnp.pad(src, (0, _EPAD - _E),
                   constant_values=_PAD_IDX).reshape(_NS, _NCHUNK, _C)
    dstp = jnp.pad(dst, (0, _EPAD - _E),
                   constant_values=_PAD_IDX).reshape(_NS, _NCHUNK, _C)
    featp = jnp.pad(features, ((0, _NP - _N), (0, 0)))
    feat3 = featp.reshape(_NP, _NC, _DH).transpose(1, 0, 2)
    out3 = _launch(feat3, srcp, dstp)
    return out3.transpose(1, 0, 2).reshape(_NP, _D)[:_N]


# speedup vs baseline: 2.7521x; 2.7521x over previous
import jax
import jax.numpy as jnp
from jax import lax
from jax.experimental import pallas as pl
from jax.experimental.pallas import tpu as pltpu
from jax.experimental.pallas import tpu_sc as plsc

_N = 10000
_E = 320000
_D = 128
_K = 10
_ALPHA = 0.1

_NC = 2
_NS = 16
_DH = _D // _NC
_NP = 10240
_RPT = _NP // _NS
_RC = 64
_NRC = _RPT // _RC
_C = 128
_NCHUNK = 160
_EPAD = _NS * _NCHUNK * _C
_PAD_IDX = _N
_L = 16


def _nr_rsqrt(x):
    i = lax.bitcast_convert_type(x, jnp.int32)
    i = jnp.int32(0x5F3759DF) - (i >> 1)
    y = lax.bitcast_convert_type(i, jnp.float32)
    for _ in range(3):
        y = y * (1.5 - 0.5 * x * y * y)
    return y


def _appnp_body(feat, srci, dsti, out, u_hbm, acc_sp, src_v, dst_v,
                y_v, buf0, buf1, buf2, buf3, zbuf,
                gs0, gs1, gs2, gs3, ss0, ss1, ss2, ss3):
    c = lax.axis_index("c")
    s = lax.axis_index("s")
    row0 = s * _RPT

    pltpu.sync_copy(srci.at[s], src_v)
    pltpu.sync_copy(dsti.at[s], dst_v)
    ubase = c * _NP

    def _offs(j, _):
        for k in range(_C // _L):
            src_v[j, pl.ds(_L * k, _L)] = src_v[j, pl.ds(_L * k, _L)] + ubase
        return 0
    lax.fori_loop(0, _NCHUNK, _offs, 0)

    def _zrow(r, _):
        z = jnp.zeros((_L,), jnp.float32)
        for k in range(_DH // _L):
            zbuf[r, pl.ds(_L * k, _L)] = z
        return 0
    lax.fori_loop(0, _RC, _zrow, 0)

    def _zacc(i, _):
        pltpu.sync_copy(zbuf, acc_sp.at[pl.ds(row0 + i * _RC, _RC)])
        return 0
    lax.fori_loop(0, _NRC, _zacc, 0)

    def _ones(r, _):
        o = jnp.full((_L,), 1.0, jnp.float32)
        for k in range(_DH // _L):
            buf0[r, pl.ds(_L * k, _L)] = o
        return 0
    lax.fori_loop(0, _C, _ones, 0)
    plsc.subcore_barrier()

    def _deg(j, _):
        pltpu.sync_copy(buf0, acc_sp.at[dst_v.at[j]], add=True)
        return 0
    lax.fori_loop(0, _NCHUNK, _deg, 0)
    plsc.subcore_barrier()

    def _ab_chunk(ci, _):
        r0 = row0 + ci * _RC
        pltpu.sync_copy(acc_sp.at[pl.ds(r0, _RC)], buf0.at[pl.ds(0, _RC)])
        pltpu.sync_copy(feat.at[c].at[pl.ds(r0, _RC)], buf1.at[pl.ds(0, _RC)])

        def _row(r, _2):
            lr = ci * _RC + r
            d = buf0[r, pl.ds(0, _L)]
            y = _nr_rsqrt(jnp.maximum(d, 1.0))
            y_v[lr, pl.ds(0, _L)] = y
            for k in range(_DH // _L):
                h0 = buf1[r, pl.ds(_L * k, _L)]
                buf0[r, pl.ds(_L * k, _L)] = y * h0
            return 0
        lax.fori_loop(0, _RC, _row, 0)
        pltpu.sync_copy(buf0.at[pl.ds(0, _RC)], u_hbm.at[pl.ds(ubase + r0, _RC)])
        pltpu.sync_copy(zbuf, acc_sp.at[pl.ds(r0, _RC)])
        return 0
    lax.fori_loop(0, _NRC, _ab_chunk, 0)
    plsc.subcore_barrier()

    bufs = (buf0, buf1, buf2, buf3)
    gss = (gs0, gs1, gs2, gs3)
    sss = (ss0, ss1, ss2, ss3)

    def _propagate():
        pltpu.async_copy(u_hbm.at[src_v.at[0]], buf0, gs0)
        pltpu.async_copy(u_hbm.at[src_v.at[1]], buf1, gs1)

        def _quad(t, _):
            for p in range(4):
                j = 4 * t + p
                b = p
                b2 = (p + 2) % 4
                pltpu.make_async_copy(u_hbm.at[src_v.at[j]], bufs[b],
                                      gss[b]).wait()
                pltpu.async_copy(bufs[b], acc_sp.at[dst_v.at[j]], sss[b],
                                 add=True)
                if p < 2:
                    @pl.when(t > 0)
                    def _w():
                        pltpu.make_async_copy(bufs[b2],
                                              acc_sp.at[dst_v.at[j - 2]],
                                              sss[b2]).wait()
                    pltpu.async_copy(u_hbm.at[src_v.at[j + 2]], bufs[b2],
                                     gss[b2])
                else:
                    pltpu.make_async_copy(bufs[b2],
                                          acc_sp.at[dst_v.at[j - 2]],
                                          sss[b2]).wait()

                    @pl.when(j + 2 < _NCHUNK)
                    def _g():
                        pltpu.async_copy(u_hbm.at[src_v.at[j + 2]], bufs[b2],
                                         gss[b2])
            return 0
        lax.fori_loop(0, _NCHUNK // 4, _quad, 0)
        pltpu.make_async_copy(buf2, acc_sp.at[dst_v.at[_NCHUNK - 2]],
                              ss2).wait()
        pltpu.make_async_copy(buf3, acc_sp.at[dst_v.at[_NCHUNK - 1]],
                              ss3).wait()

    def _dense():
        def _chunk(ci, _):
            r0 = row0 + ci * _RC
            pltpu.sync_copy(acc_sp.at[pl.ds(r0, _RC)], buf0.at[pl.ds(0, _RC)])
            pltpu.sync_copy(feat.at[c].at[pl.ds(r0, _RC)],
                            buf1.at[pl.ds(0, _RC)])

            def _row(r, _2):
                lr = ci * _RC + r
                y = y_v[lr, pl.ds(0, _L)]
                a = (1.0 - _ALPHA) * (y * y)
                for k in range(_DH // _L):
                    acc16 = buf0[r, pl.ds(_L * k, _L)]
                    b16 = _ALPHA * (y * buf1[r, pl.ds(_L * k, _L)])
                    buf0[r, pl.ds(_L * k, _L)] = a * acc16 + b16
                return 0
            lax.fori_loop(0, _RC, _row, 0)
            pltpu.sync_copy(buf0.at[pl.ds(0, _RC)],
                            u_hbm.at[pl.ds(ubase + r0, _RC)])
            pltpu.sync_copy(zbuf, acc_sp.at[pl.ds(r0, _RC)])
            return 0
        lax.fori_loop(0, _NRC, _chunk, 0)

    def _layer(l, _):
        _propagate()
        plsc.subcore_barrier()
        _dense()
        plsc.subcore_barrier()
        return 0
    lax.fori_loop(0, _K - 1, _layer, 0)

    _propagate()
    plsc.subcore_barrier()

    def _out_chunk(ci, _):
        r0 = row0 + ci * _RC
        pltpu.sync_copy(acc_sp.at[pl.ds(r0, _RC)], buf0.at[pl.ds(0, _RC)])
        pltpu.sync_copy(feat.at[c].at[pl.ds(r0, _RC)], buf1.at[pl.ds(0, _RC)])

        def _row(r, _2):
            lr = ci * _RC + r
            y = y_v[lr, pl.ds(0, _L)]
            for k in range(_DH // _L):
                acc16 = buf0[r, pl.ds(_L * k, _L)]
                h016 = buf1[r, pl.ds(_L * k, _L)]
                buf0[r, pl.ds(_L * k, _L)] = ((1.0 - _ALPHA) * (y * acc16)
                                              + _ALPHA * h016)
            return 0
        lax.fori_loop(0, _RC, _row, 0)
        pltpu.sync_copy(buf0.at[pl.ds(0, _RC)], out.at[c].at[pl.ds(r0, _RC)])
        return 0
    lax.fori_loop(0, _NRC, _out_chunk, 0)


_launch = pl.kernel(
    _appnp_body,
    out_type=jax.ShapeDtypeStruct((_NC, _NP, _DH), jnp.float32),
    mesh=plsc.VectorSubcoreMesh(core_axis_name="c", subcore_axis_name="s"),
    compiler_params=pltpu.CompilerParams(use_tc_tiling_on_sc=False),
    scratch_types=[
        pltpu.HBM((_NC * _NP, _DH), jnp.float32),
        pltpu.VMEM_SHARED((_NP, _DH), jnp.float32),
        pltpu.VMEM((_NCHUNK, _C), jnp.int32),
        pltpu.VMEM((_NCHUNK, _C), jnp.int32),
        pltpu.VMEM((_RPT, _L), jnp.float32),
        pltpu.VMEM((_C, _DH), jnp.float32),
        pltpu.VMEM((_C, _DH), jnp.float32),
        pltpu.VMEM((_C, _DH), jnp.float32),
        pltpu.VMEM((_C, _DH), jnp.float32),
        pltpu.VMEM((_RC, _DH), jnp.float32),
        pltpu.SemaphoreType.DMA,
        pltpu.SemaphoreType.DMA,
        pltpu.SemaphoreType.DMA,
        pltpu.SemaphoreType.DMA,
        pltpu.SemaphoreType.DMA,
        pltpu.SemaphoreType.DMA,
        pltpu.SemaphoreType.DMA,
        pltpu.SemaphoreType.DMA,
    ],
)


def kernel(features, edge_index):
    src = edge_index[0]
    dst = edge_index[1]
    srcp = jnp.pad(src, (0, _EPAD - _E),
                   constant_values=_PAD_IDX).reshape(_NS, _NCHUNK, _C)
    dstp = jnp.pad(dst, (0, _EPAD - _E),
                   constant_values=_PAD_IDX).reshape(_NS, _NCHUNK, _C)
    featp = jnp.pad(features, ((0, _NP - _N), (0, 0)))
    feat3 = featp.reshape(_NP, _NC, _DH).transpose(1, 0, 2)
    out3 = _launch(feat3, srcp, dstp)
    return out3.transpose(1, 0, 2).reshape(_NP, _D)[:_N]
